# baseline (speedup 1.0000x reference)
import functools
import jax
import jax.numpy as jnp
from jax.experimental import pallas as pl
from jax.experimental.pallas import tpu as pltpu


def _gcn_body(nblk, br, bc, x_hbm, w_ref, b_ref, adj_hbm, out_ref,
              x_ref, s_ref, tail_ref, buf_ref, sems, x_sem, tail_sem):
    nbuf = buf_ref.shape[0]
    n = adj_hbm.shape[0]
    tail = n - bc

    def start_copy(i, slot):
        pltpu.make_async_copy(
            adj_hbm.at[pl.ds(i * br, br), pl.ds(0, bc)],
            buf_ref.at[slot],
            sems.at[slot],
        ).start()

    x_copy = pltpu.make_async_copy(x_hbm, x_ref, x_sem)
    x_copy.start()
    tail_copy = pltpu.make_async_copy(
        adj_hbm.at[:, pl.ds(bc, tail)], tail_ref, tail_sem)
    tail_copy.start()

    for k in range(min(nbuf, nblk)):
        start_copy(k, k)

    x_copy.wait()
    s_ref[...] = jnp.dot(x_ref[...], w_ref[...],
                         preferred_element_type=jnp.float32)

    def loop(i, carry):
        slot = jax.lax.rem(i, nbuf)
        pltpu.make_async_copy(
            adj_hbm.at[pl.ds(i * br, br), pl.ds(0, bc)],
            buf_ref.at[slot],
            sems.at[slot],
        ).wait()
        acc = jnp.dot(buf_ref[slot], s_ref[pl.ds(0, bc), :],
                      preferred_element_type=jnp.float32)
        out_ref[pl.ds(i * br, br), :] = acc

        @pl.when(i + nbuf < nblk)
        def _():
            start_copy(i + nbuf, slot)

        return carry

    jax.lax.fori_loop(0, nblk, loop, 0)

    tail_copy.wait()
    e = jnp.dot(tail_ref[...], s_ref[pl.ds(bc, tail), :],
                preferred_element_type=jnp.float32)
    out_ref[...] = jnp.maximum(out_ref[...] + e + b_ref[...], 0.0)


def kernel(x, adj, W, b):
    N, F = x.shape
    H = W.shape[1]

    BR = 200
    NBUF = 4
    BC = (N // 128) * 128
    nblk = N // BR

    out = pl.pallas_call(
        functools.partial(_gcn_body, nblk, BR, BC),
        in_specs=[
            pl.BlockSpec(memory_space=pltpu.HBM),
            pl.BlockSpec(memory_space=pltpu.VMEM),
            pl.BlockSpec(memory_space=pltpu.VMEM),
            pl.BlockSpec(memory_space=pltpu.HBM),
        ],
        out_specs=pl.BlockSpec(memory_space=pltpu.VMEM),
        out_shape=jax.ShapeDtypeStruct((N, H), jnp.float32),
        scratch_shapes=[
            pltpu.VMEM((N, F), jnp.float32),
            pltpu.VMEM((N, H), jnp.float32),
            pltpu.VMEM((N, N - BC), jnp.float32),
            pltpu.VMEM((NBUF, BR, BC), jnp.float32),
            pltpu.SemaphoreType.DMA((NBUF,)),
            pltpu.SemaphoreType.DMA,
            pltpu.SemaphoreType.DMA,
        ],
    )(x, W, b.reshape(1, H), adj)
    return out
